# Initial kernel scaffold; baseline (speedup 1.0000x reference)
import jax
import jax.numpy as jnp
from jax.experimental import pallas as pl


def kernel(base_inputs, current_step):
    raise NotImplementedError("write your pallas kernel here")


# speedup vs baseline: 1.7689x; 1.7689x over previous
import functools

import jax
import jax.numpy as jnp
from jax import lax
from jax.experimental import pallas as pl
from jax.experimental.pallas import tpu as pltpu, tpu_sc as plsc

NUM_STREAMS = 8
B = 128
L = 32768
LANES = 16
NBUF = 3

_info = plsc.get_sparse_core_info()
NC, NS = _info.num_cores, _info.num_subcores
NW = NC * NS
ROWS_PER_WORKER_PER_STREAM = B // NW


def _fixup(buf_ref, add_v, stream_idx):
    st = (stream_idx + 1) * 10
    count = (L + st - 1) // st
    chunks = (count + LANES - 1) // LANES
    iota = lax.iota(jnp.int32, LANES)

    def body(c, carry):
        idx = (c * LANES + iota) * st
        m = idx < L
        g = plsc.load_gather(buf_ref, [idx], mask=m)
        y = g + add_v
        y = jnp.where(y >= 4096.0, y - 4096.0, y)
        plsc.store_scatter(buf_ref, [idx], y, mask=m)
        return carry

    lax.fori_loop(0, chunks, body, 0)


def _make_sc_kernel():
    mesh = plsc.VectorSubcoreMesh(core_axis_name="c", subcore_axis_name="s")

    @functools.partial(
        pl.kernel,
        mesh=mesh,
        compiler_params=pltpu.CompilerParams(
            needs_layout_passes=False, use_tc_tiling_on_sc=False),
        out_type=jax.ShapeDtypeStruct((NUM_STREAMS * B, L), jnp.float32),
        scratch_types=[
            pltpu.VMEM((L,), jnp.float32),
            pltpu.VMEM((L,), jnp.float32),
            pltpu.VMEM((L,), jnp.float32),
            pltpu.VMEM((NUM_STREAMS * LANES,), jnp.float32),
            pltpu.SemaphoreType.DMA,
            pltpu.SemaphoreType.DMA,
            pltpu.SemaphoreType.DMA,
            pltpu.SemaphoreType.DMA,
            pltpu.SemaphoreType.DMA,
            pltpu.SemaphoreType.DMA,
        ],
    )
    def sc_kernel(base_hbm, adds_hbm, out_hbm, buf0, buf1, buf2, adds_v, *sems):
        bufs = (buf0, buf1, buf2)
        sem_in = sems[:NBUF]
        sem_out = sems[NBUF:]
        wid = lax.axis_index("s") * NC + lax.axis_index("c")

        pltpu.sync_copy(adds_hbm, adds_v)

        steps = [(s, i) for s in range(NUM_STREAMS)
                 for i in range(ROWS_PER_WORKER_PER_STREAM)]
        n = len(steps)
        in_d = [None] * n
        out_d = [None] * n

        def start_in(t):
            _, i = steps[t]
            row = i * NW + wid
            in_d[t] = pltpu.async_copy(
                base_hbm.at[row], bufs[t % NBUF], sem_in[t % NBUF])

        start_in(0)
        for t in range(n):
            s, i = steps[t]
            if t + 1 < n:
                if t - (NBUF - 1) >= 0:
                    out_d[t - (NBUF - 1)].wait()
                start_in(t + 1)
            in_d[t].wait()
            if s > 0:
                add_v = adds_v[pl.ds(s * LANES, LANES)]
                _fixup(bufs[t % NBUF], add_v, s)
            orow = s * B + i * NW + wid
            out_d[t] = pltpu.async_copy(
                bufs[t % NBUF], out_hbm.at[orow], sem_out[t % NBUF])
        for t in range(n - (NBUF - 1), n):
            out_d[t].wait()

    return sc_kernel


_sc_kernel = _make_sc_kernel()


def kernel(base_inputs, current_step):
    active = (jnp.asarray(current_step) > 0).astype(jnp.float32)
    adds = (jnp.arange(NUM_STREAMS, dtype=jnp.float32)[:, None] * active
            * jnp.ones((1, LANES), jnp.float32)).reshape(-1)
    return _sc_kernel(base_inputs, adds)
